# Initial kernel scaffold; baseline (speedup 1.0000x reference)
import jax
import jax.numpy as jnp
from jax.experimental import pallas as pl


def kernel(poi_path, W0, W1, W2, W3):
    raise NotImplementedError("write your pallas kernel here")


# speedup vs baseline: 10.3712x; 10.3712x over previous
import functools

import jax
import jax.numpy as jnp
from jax import lax
from jax.experimental import pallas as pl
from jax.experimental.pallas import tpu as pltpu
from jax.experimental.pallas import tpu_sc as plsc

EMB = 32
NT = 4
SUB = 128
CHUNK = 512
NSUB = CHUNK // SUB


def _make_lookup(n_rows, table_rows):
    info = plsc.get_sparse_core_info()
    nw = info.num_cores * info.num_subcores
    n_per_w = n_rows // nw
    n_chunks = n_per_w // CHUNK
    assert n_per_w * nw == n_rows and n_chunks * CHUNK == n_per_w

    mesh = plsc.VectorSubcoreMesh(core_axis_name="c", subcore_axis_name="s")

    @functools.partial(
        pl.kernel,
        out_type=jax.ShapeDtypeStruct((n_rows, EMB), jnp.float32),
        mesh=mesh,
        scratch_types=[
            pltpu.VMEM((NT, NSUB, SUB), jnp.int32),
            pltpu.VMEM((NT, CHUNK, EMB), jnp.float32),
            pltpu.VMEM((CHUNK, EMB), jnp.float32),
            pltpu.SemaphoreType.DMA,
        ],
        compiler_params=pltpu.CompilerParams(use_tc_tiling_on_sc=False),
    )
    def lookup(idx_hbm, w0, w1, w2, w3, out_hbm, idx_v, rows_v, out_v, sem):
        tables = (w0, w1, w2, w3)
        wid = lax.axis_index("s") * info.num_cores + lax.axis_index("c")
        base_row = wid * (n_per_w // SUB)

        def chunk_body(k, carry):
            irow = base_row + k * NSUB
            off = irow * SUB
            for t in range(NT):
                pltpu.sync_copy(idx_hbm.at[t, pl.ds(irow, NSUB)], idx_v.at[t])
            copies = []
            for t in range(NT):
                for m in range(NSUB):
                    copies.append(pltpu.async_copy(
                        tables[t].at[idx_v.at[t, m]],
                        rows_v.at[t, pl.ds(m * SUB, SUB)],
                        sem))
            for c in copies:
                c.wait()

            def row_body(j, carry2):
                for h in (0, EMB // 2):
                    d = pl.ds(h, EMB // 2)
                    s01 = rows_v[0, j, d] + rows_v[1, j, d]
                    s23 = rows_v[2, j, d] + rows_v[3, j, d]
                    out_v[j, d] = (s01 + s23) * jnp.float32(0.25)
                return carry2

            lax.fori_loop(0, CHUNK, row_body, 0, unroll=2)
            pltpu.sync_copy(out_v, out_hbm.at[pl.ds(off, CHUNK)])
            return carry

        lax.fori_loop(0, n_chunks, chunk_body, 0)

    return lookup


def kernel(poi_path, W0, W1, W2, W3):
    b, h, nt = poi_path.shape
    n = b * h
    idx_t = poi_path.reshape(n, nt).T.reshape(nt, n // SUB, SUB)
    out = _make_lookup(n, W0.shape[0])(idx_t, W0, W1, W2, W3)
    return out.reshape(b, h, EMB)


# speedup vs baseline: 12.8656x; 1.2405x over previous
import functools

import jax
import jax.numpy as jnp
from jax import lax
from jax.experimental import pallas as pl
from jax.experimental.pallas import tpu as pltpu
from jax.experimental.pallas import tpu_sc as plsc

EMB = 32
NT = 4
SUB = 128
CHUNK = 256
NSUB = CHUNK // SUB


def _make_lookup(n_rows):
    info = plsc.get_sparse_core_info()
    nw = info.num_cores * info.num_subcores
    n_per_w = n_rows // nw
    n_chunks = n_per_w // CHUNK
    assert n_per_w * nw == n_rows and n_chunks * CHUNK == n_per_w
    assert n_chunks % 2 == 0

    mesh = plsc.VectorSubcoreMesh(core_axis_name="c", subcore_axis_name="s")

    @functools.partial(
        pl.kernel,
        out_type=jax.ShapeDtypeStruct((n_rows, EMB), jnp.float32),
        mesh=mesh,
        scratch_types=[
            pltpu.VMEM((2, NT, NSUB, SUB), jnp.int32),
            pltpu.VMEM((2, NT, CHUNK, EMB), jnp.float32),
            pltpu.VMEM((2, CHUNK, EMB), jnp.float32),
            pltpu.SemaphoreType.DMA,
            pltpu.SemaphoreType.DMA,
            pltpu.SemaphoreType.DMA,
            pltpu.SemaphoreType.DMA,
            pltpu.SemaphoreType.DMA,
            pltpu.SemaphoreType.DMA,
        ],
        compiler_params=pltpu.CompilerParams(use_tc_tiling_on_sc=False),
    )
    def lookup(idx_hbm, w0, w1, w2, w3, out_hbm,
               idx_v, rows_v, out_v, gs0, gs1, is0, is1, os0, os1):
        tables = (w0, w1, w2, w3)
        gsem = (gs0, gs1)
        isem = (is0, is1)
        osem = (os0, os1)
        wid = lax.axis_index("s") * info.num_cores + lax.axis_index("c")
        base_irow = wid * (n_per_w // SUB)

        def idx_copies(k, sp):
            irow = base_irow + k * NSUB
            return [pltpu.make_async_copy(
                idx_hbm.at[t, pl.ds(irow, NSUB)], idx_v.at[sp, t], isem[sp])
                for t in range(NT)]

        def gather_copies(k, sp):
            del k
            return [pltpu.make_async_copy(
                tables[t].at[idx_v.at[sp, t, m]],
                rows_v.at[sp, t, pl.ds(m * SUB, SUB)], gsem[sp])
                for t in range(NT) for m in range(NSUB)]

        def out_copy(k, sp):
            off = (base_irow + k * NSUB) * SUB
            return pltpu.make_async_copy(
                out_v.at[sp], out_hbm.at[pl.ds(off, CHUNK)], osem[sp])

        for c in idx_copies(0, 0):
            c.start()
        for c in idx_copies(0, 0):
            c.wait()
        for c in gather_copies(0, 0):
            c.start()
        for c in idx_copies(1, 1):
            c.start()

        def pair_body(kk, carry):
            for s in (0, 1):
                k = 2 * kk + s
                sn = 1 - s
                for c in gather_copies(k, s):
                    c.wait()
                @pl.when(k + 2 < n_chunks)
                def _prefetch_idx():
                    for c in idx_copies(k + 2, s):
                        c.start()

                @pl.when(k + 1 < n_chunks)
                def _fire_next():
                    for c in idx_copies(k + 1, sn):
                        c.wait()
                    for c in gather_copies(k + 1, sn):
                        c.start()

                @pl.when(k >= 2)
                def _reclaim_out():
                    out_copy(k - 2, s).wait()

                def row_body(j, carry2):
                    for h in (0, EMB // 2):
                        d = pl.ds(h, EMB // 2)
                        s01 = rows_v[s, 0, j, d] + rows_v[s, 1, j, d]
                        s23 = rows_v[s, 2, j, d] + rows_v[s, 3, j, d]
                        out_v[s, j, d] = (s01 + s23) * jnp.float32(0.25)
                    return carry2

                lax.fori_loop(0, CHUNK, row_body, 0, unroll=8)
                out_copy(k, s).start()
            return carry

        lax.fori_loop(0, n_chunks // 2, pair_body, 0)
        out_copy(n_chunks - 2, 0).wait()
        out_copy(n_chunks - 1, 1).wait()

    return lookup


def kernel(poi_path, W0, W1, W2, W3):
    b, h, nt = poi_path.shape
    n = b * h
    idx_t = poi_path.reshape(n, nt).T.reshape(nt, n // SUB, SUB)
    out = _make_lookup(n)(idx_t, W0, W1, W2, W3)
    return out.reshape(b, h, EMB)
